# Initial kernel scaffold; baseline (speedup 1.0000x reference)
import jax
import jax.numpy as jnp
from jax.experimental import pallas as pl


def kernel(x):
    raise NotImplementedError("write your pallas kernel here")


# speedup vs baseline: 2.3373x; 2.3373x over previous
import jax
import jax.numpy as jnp
from jax.experimental import pallas as pl
from jax.experimental.pallas import tpu as pltpu

_B = 4


def _eeg_body(x_ref, o_ref):
    x = x_ref[...]
    xc = x - jnp.mean(x, axis=1, keepdims=True)
    m = jnp.mean(xc, axis=2, keepdims=True)
    d = xc - m
    var = jnp.mean(d * d, axis=2, keepdims=True)
    scale = jnp.where(var > 0.0, jax.lax.rsqrt(var), 1.0)
    o_ref[...] = d * scale


def kernel(x):
    batch, ch, t = x.shape
    grid = (batch // _B,)
    return pl.pallas_call(
        _eeg_body,
        grid=grid,
        in_specs=[pl.BlockSpec((_B, ch, t), lambda i: (i, 0, 0))],
        out_specs=pl.BlockSpec((_B, ch, t), lambda i: (i, 0, 0)),
        out_shape=jax.ShapeDtypeStruct(x.shape, x.dtype),
        compiler_params=pltpu.CompilerParams(
            dimension_semantics=("parallel",),
            vmem_limit_bytes=56 * 1024 * 1024,
        ),
        name="eeg_preprocess",
    )(x)


# speedup vs baseline: 2.4072x; 1.0299x over previous
import jax
import jax.numpy as jnp
from jax.experimental import pallas as pl
from jax.experimental.pallas import tpu as pltpu

_B = 8


def _eeg_body(x_ref, o_ref):
    x = x_ref[...]
    xc = x - jnp.mean(x, axis=1, keepdims=True)
    m = jnp.mean(xc, axis=2, keepdims=True)
    d = xc - m
    var = jnp.mean(d * d, axis=2, keepdims=True)
    scale = jnp.where(var > 0.0, jax.lax.rsqrt(var), 1.0)
    o_ref[...] = d * scale


def kernel(x):
    batch, ch, t = x.shape
    grid = (batch // _B,)
    return pl.pallas_call(
        _eeg_body,
        grid=grid,
        in_specs=[pl.BlockSpec((_B, ch, t), lambda i: (i, 0, 0))],
        out_specs=pl.BlockSpec((_B, ch, t), lambda i: (i, 0, 0)),
        out_shape=jax.ShapeDtypeStruct(x.shape, x.dtype),
        compiler_params=pltpu.CompilerParams(
            dimension_semantics=("parallel",),
            vmem_limit_bytes=56 * 1024 * 1024,
        ),
        name="eeg_preprocess",
    )(x)
